# baseline (speedup 1.0000x reference)
import functools

import jax
import jax.numpy as jnp
import numpy as np
from jax.experimental import pallas as pl

N_NODES = 10000
HIDDEN = 128
NEG_BIG = -1e30

EDGE_BLK = 512
NODE_BLK = 1000


def _k0_body(vp_ref, wp_ref, bp_ref, wd_ref, t_ref, s1_ref):
    t0 = jnp.dot(vp_ref[...], wp_ref[...], preferred_element_type=jnp.float32)
    t0 = t0 + bp_ref[...]
    t_ref[...] = t0
    s1_ref[...] = jnp.dot(t0, wd_ref[...], preferred_element_type=jnp.float32)


def _node_init(verts_pad, W_pos_pad, b_pos, Wd0):
    n = verts_pad.shape[0]
    grid = (n // NODE_BLK,)
    return pl.pallas_call(
        _k0_body,
        grid=grid,
        in_specs=[
            pl.BlockSpec((NODE_BLK, 8), lambda i: (i, 0)),
            pl.BlockSpec((8, 256), lambda i: (0, 0)),
            pl.BlockSpec((1, 256), lambda i: (0, 0)),
            pl.BlockSpec((256, 128), lambda i: (0, 0)),
        ],
        out_specs=[
            pl.BlockSpec((NODE_BLK, 256), lambda i: (i, 0)),
            pl.BlockSpec((NODE_BLK, 128), lambda i: (i, 0)),
        ],
        out_shape=[
            jax.ShapeDtypeStruct((n, 256), jnp.float32),
            jax.ShapeDtypeStruct((n, 128), jnp.float32),
        ],
    )(verts_pad, W_pos_pad, b_pos, Wd0)


def _k1_body(seg_ref, s1p_ref, b1_ref, p_ref, wd_ref, t_ref, s1_ref):
    net = seg_ref[...] + b1_ref[...] + s1p_ref[...]
    net = jnp.where(net > NEG_BIG, net, 0.0)
    pool = jnp.max(net, axis=1, keepdims=True)
    blk = net.shape[0]
    t = jnp.concatenate(
        [net, pool, p_ref[...][:, :3], jnp.zeros((blk, 12), jnp.float32)], axis=1)
    t_ref[...] = t
    s1_ref[...] = jnp.dot(t, wd_ref[...], preferred_element_type=jnp.float32)


def _node_stage(seg, s1_prev, b1_prev, verts_pad, Wd_rec):
    n = seg.shape[0]
    grid = (n // NODE_BLK,)
    return pl.pallas_call(
        _k1_body,
        grid=grid,
        in_specs=[
            pl.BlockSpec((NODE_BLK, 128), lambda i: (i, 0)),
            pl.BlockSpec((NODE_BLK, 128), lambda i: (i, 0)),
            pl.BlockSpec((1, 128), lambda i: (0, 0)),
            pl.BlockSpec((NODE_BLK, 8), lambda i: (i, 0)),
            pl.BlockSpec((144, 128), lambda i: (0, 0)),
        ],
        out_specs=[
            pl.BlockSpec((NODE_BLK, 144), lambda i: (i, 0)),
            pl.BlockSpec((NODE_BLK, 128), lambda i: (i, 0)),
        ],
        out_shape=[
            jax.ShapeDtypeStruct((n, 144), jnp.float32),
            jax.ShapeDtypeStruct((n, 128), jnp.float32),
        ],
    )(seg, s1_prev, b1_prev, verts_pad, Wd_rec)


def _k2_body(gr_ref, gc_ref, wi_ref, w2_ref, b0_ref, m_ref):
    gr = gr_ref[...]
    gc = gc_ref[...]
    a = jnp.concatenate(
        [jnp.maximum(gr, 0.0), gr, gc, jnp.abs(gc - gr)], axis=1)
    inner = jnp.dot(a, wi_ref[...], preferred_element_type=jnp.float32)
    inner = jnp.maximum(inner + b0_ref[...], 0.0)
    a2 = jnp.concatenate([inner, gc], axis=1)
    m_ref[...] = jnp.dot(a2, w2_ref[...], preferred_element_type=jnp.float32)


def _edge_stage(GR, GC, Winner, W2, b0):
    ep, rec = GR.shape
    grid = (ep // EDGE_BLK,)
    return pl.pallas_call(
        _k2_body,
        grid=grid,
        in_specs=[
            pl.BlockSpec((EDGE_BLK, rec), lambda i: (i, 0)),
            pl.BlockSpec((EDGE_BLK, rec), lambda i: (i, 0)),
            pl.BlockSpec((4 * rec, 128), lambda i: (0, 0)),
            pl.BlockSpec((rec + 128, 128), lambda i: (0, 0)),
            pl.BlockSpec((1, 128), lambda i: (0, 0)),
        ],
        out_specs=pl.BlockSpec((EDGE_BLK, 128), lambda i: (i, 0)),
        out_shape=jax.ShapeDtypeStruct((ep, 128), jnp.float32),
    )(GR, GC, Winner, W2, b0)


def _kf_body(seg_ref, s1_ref, b1_ref, wc_ref, bc_ref, c_ref):
    net = seg_ref[...] + b1_ref[...] + s1_ref[...]
    net = jnp.where(net > NEG_BIG, net, 0.0)
    net = jnp.maximum(net, 0.0)
    c_ref[...] = jnp.dot(net, wc_ref[...],
                         preferred_element_type=jnp.float32) + bc_ref[...]


def _final_stage(seg, s1_prev, b1_prev, W_c, b_c):
    n = seg.shape[0]
    grid = (n // NODE_BLK,)
    return pl.pallas_call(
        _kf_body,
        grid=grid,
        in_specs=[
            pl.BlockSpec((NODE_BLK, 128), lambda i: (i, 0)),
            pl.BlockSpec((NODE_BLK, 128), lambda i: (i, 0)),
            pl.BlockSpec((1, 128), lambda i: (0, 0)),
            pl.BlockSpec((128, 128), lambda i: (0, 0)),
            pl.BlockSpec((1, 128), lambda i: (0, 0)),
        ],
        out_specs=pl.BlockSpec((NODE_BLK, 128), lambda i: (i, 0)),
        out_shape=jax.ShapeDtypeStruct((n, 128), jnp.float32),
    )(seg, s1_prev, b1_prev, W_c, b_c)


def _to_rec(M):
    return jnp.concatenate([
        M[:128],
        jnp.sum(M[128:256], axis=0, keepdims=True),
        M[256:259],
        jnp.zeros((12, 128), jnp.float32),
    ], axis=0)


def _pack_layer(blk, rec):
    d2 = 256 if rec == 256 else 259
    W0, b0, W1, b1, Ws = blk["W0"], blk["b0"], blk["W1"], blk["b1"], blk["Ws"]
    W0a, W0b = W0[:d2], W0[d2:]
    Wsa, Wsb = Ws[:d2], Ws[d2:]
    conv = (lambda m: m) if rec == 256 else _to_rec
    W0a_r, W0b_r = conv(W0a), conv(W0b)
    Winner = jnp.concatenate(
        [W0a_r, -0.5 * W0b_r, 0.5 * W0b_r, 0.5 * W0b_r], axis=0)
    W2 = jnp.concatenate([W1, conv(Wsb)], axis=0)
    Wd = conv(Wsa - Wsb)
    return Winner, W2, Wd, b0[None, :], b1[None, :]


def kernel(verts, faces, W_pos, b_pos, blocks, W_c, b_c):
    n = verts.shape[0]
    e = jnp.concatenate(
        [faces[:, [0, 1]], faces[:, [0, 2]], faces[:, [1, 2]]], axis=0)
    e = jnp.sort(e, axis=1)
    row = jnp.concatenate([e[:, 0], e[:, 1]])
    col = jnp.concatenate([e[:, 1], e[:, 0]])
    perm = jnp.argsort(row)
    srow = row[perm]
    scol = col[perm]
    E = srow.shape[0]
    EP = ((E + EDGE_BLK - 1) // EDGE_BLK) * EDGE_BLK

    verts_pad = jnp.pad(verts, ((0, 0), (0, 5)))
    W_pos_pad = jnp.pad(W_pos, ((0, 5), (0, 0)))

    packs = [_pack_layer(blk, 256 if i == 0 else 144)
             for i, blk in enumerate(blocks)]

    T, s1 = _node_init(verts_pad, W_pos_pad, b_pos[None, :], packs[0][2])
    for l in range(5):
        Winner, W2, _, b0, b1 = packs[l]
        GR = jnp.pad(jnp.take(T, srow, axis=0), ((0, EP - E), (0, 0)))
        GC = jnp.pad(jnp.take(T, scol, axis=0), ((0, EP - E), (0, 0)))
        M = _edge_stage(GR, GC, Winner, W2, b0)
        seg = jax.ops.segment_max(M[:E], srow, num_segments=n)
        if l < 4:
            T, s1 = _node_stage(seg, s1, b1, verts_pad, packs[l + 1][2])
        else:
            out = _final_stage(seg, s1, b1, W_c, b_c[None, :])
    return out
